# baseline (device time: 99223 ns/iter reference)
import jax
import jax.numpy as jnp
from jax import lax
from jax.experimental import pallas as pl
from jax.experimental.pallas import tpu as pltpu

N_DEV = 8
N_LAYER = 3


def kernel(x, Win0, Wout0, Win1, Wout1, Win2, Wout2):
    m, d = x.shape
    dh = Win0.shape[1]

    def body(x_ref, win0, wout0, win1, wout1, win2, wout2, out_ref,
             winbuf, woutbuf, xg, psend, prec, own, xbuf,
             ag_s, ag_r, rs_s, rs_r, wsem):
        my = lax.axis_index("i")
        wins = [win0, win1, win2]
        wouts = [wout0, wout1, wout2]

        wcp = []
        for l in range(2):
            cp_in = pltpu.make_async_copy(wins[l], winbuf.at[l], wsem.at[2 * l])
            cp_out = pltpu.make_async_copy(wouts[l], woutbuf.at[l],
                                           wsem.at[2 * l + 1])
            cp_in.start()
            cp_out.start()
            wcp.append((cp_in, cp_out))

        barrier = pltpu.get_barrier_semaphore()
        for k in range(1, N_DEV):
            pl.semaphore_signal(
                barrier, inc=1,
                device_id=(lax.rem(my + k, N_DEV),),
                device_id_type=pl.DeviceIdType.MESH)
        pl.semaphore_wait(barrier, N_DEV - 1)

        my_slot = pl.ds(my * m, m)

        for l in range(N_LAYER):
            par = l % 2
            xin = x_ref if l == 0 else xbuf.at[(l - 1) % 2]

            def matmul(xb, _l=l):
                h = jnp.maximum(
                    jnp.dot(xb, winbuf[_l % 2],
                            preferred_element_type=jnp.float32),
                    0.0)
                return jnp.dot(h, woutbuf[_l % 2],
                               preferred_element_type=jnp.float32)

            ag = []
            for k in range(1, N_DEV):
                rd = pltpu.make_async_remote_copy(
                    src_ref=xin,
                    dst_ref=xg.at[pl.ds(par * N_DEV * m + my * m, m), :],
                    send_sem=ag_s.at[l, k - 1], recv_sem=ag_r.at[l, k - 1],
                    device_id=(lax.rem(my + k, N_DEV),),
                    device_id_type=pl.DeviceIdType.MESH)
                rd.start()
                ag.append(rd)

            wcp[l][0].wait()
            wcp[l][1].wait()
            own[...] = matmul(xin[...])

            rs = []
            for k in range(1, N_DEV):
                ag[k - 1].wait_recv()
                blk = pl.ds(par * N_DEV * m
                            + lax.rem(my - k + N_DEV, N_DEV) * m, m)
                po = par * (N_DEV - 1) * m + (k - 1) * m
                psend[po:po + m, :] = matmul(xg[blk, :])
                rd = pltpu.make_async_remote_copy(
                    src_ref=psend.at[po:po + m, :],
                    dst_ref=prec.at[po:po + m, :],
                    send_sem=rs_s.at[l, k - 1], recv_sem=rs_r.at[l, k - 1],
                    device_id=(lax.rem(my - k + N_DEV, N_DEV),),
                    device_id_type=pl.DeviceIdType.MESH)
                rd.start()
                rs.append(rd)

            if l == 0:
                cp_in = pltpu.make_async_copy(wins[2], winbuf.at[0],
                                              wsem.at[4])
                cp_out = pltpu.make_async_copy(wouts[2], woutbuf.at[0],
                                               wsem.at[5])
                cp_in.start()
                cp_out.start()
                wcp.append((cp_in, cp_out))

            for rd in rs:
                rd.wait_recv()
            pb = par * (N_DEV - 1) * m
            s01 = prec[pb:pb + m, :] + prec[pb + m:pb + 2 * m, :]
            s23 = prec[pb + 2 * m:pb + 3 * m, :] + prec[pb + 3 * m:pb + 4 * m, :]
            s45 = prec[pb + 4 * m:pb + 5 * m, :] + prec[pb + 5 * m:pb + 6 * m, :]
            s67 = prec[pb + 6 * m:pb + 7 * m, :] + own[...]
            result = (s01 + s23) + (s45 + s67)
            if l == N_LAYER - 1:
                out_ref[...] = result
            else:
                xbuf[l % 2, :, :] = result

            for rd in ag + rs:
                rd.wait_send()

    vmem = pl.BlockSpec(memory_space=pltpu.VMEM)
    anymem = pl.BlockSpec(memory_space=pltpu.MemorySpace.HBM)
    return pl.pallas_call(
        body,
        out_shape=jax.ShapeDtypeStruct((m, d), jnp.float32),
        in_specs=[vmem] + [anymem] * 6,
        out_specs=vmem,
        scratch_shapes=[
            pltpu.VMEM((2, d, dh), jnp.float32),
            pltpu.VMEM((2, dh, d), jnp.float32),
            pltpu.VMEM((2 * N_DEV * m, d), jnp.float32),
            pltpu.VMEM((2 * (N_DEV - 1) * m, d), jnp.float32),
            pltpu.VMEM((2 * (N_DEV - 1) * m, d), jnp.float32),
            pltpu.VMEM((m, d), jnp.float32),
            pltpu.VMEM((2, m, d), jnp.float32),
            pltpu.SemaphoreType.DMA((N_LAYER, N_DEV - 1)),
            pltpu.SemaphoreType.DMA((N_LAYER, N_DEV - 1)),
            pltpu.SemaphoreType.DMA((N_LAYER, N_DEV - 1)),
            pltpu.SemaphoreType.DMA((N_LAYER, N_DEV - 1)),
            pltpu.SemaphoreType.DMA((6,)),
        ],
        compiler_params=pltpu.CompilerParams(
            collective_id=0, vmem_limit_bytes=60 * 1024 * 1024),
    )(x, Win0, Wout0, Win1, Wout1, Win2, Wout2)
